# baseline (speedup 1.0000x reference)
import math

import jax
import jax.numpy as jnp
import numpy as np
from jax.experimental import pallas as pl
from jax.experimental.pallas import tpu as pltpu

_D_MODEL = 768
_GLOVE_DIM = 300
_SEQ = 200
_TI = 8
_VT = 10000
_VOCAB = 400000


def _pe_const():
    position = np.arange(0, _SEQ, dtype=np.float32)[:, None]
    div_term = np.exp(
        np.arange(0, _D_MODEL, 2, dtype=np.float32)
        * (-math.log(10000.0) / _D_MODEL)
    )
    pe = np.zeros((_SEQ, _D_MODEL), dtype=np.float32)
    pe[:, 0::2] = np.sin(position * div_term)
    pe[:, 1::2] = np.cos(position * div_term)
    return pe


_PE = _pe_const()


def _gather_body(toks_ref, glove_ref, vec_ref):
    k = pl.program_id(0)
    base = k * _VT
    if k is not None:
        pass


def _fused_body(vec_ref, w_ref, b_ref, pe_ref, out_ref, y_ref):
    i = pl.program_id(0)

    @pl.when(i == 0)
    def _():
        y_ref[...] = (
            jnp.dot(vec_ref[...], w_ref[...], preferred_element_type=jnp.float32)
            + b_ref[...]
        )

    out_ref[...] = y_ref[...][None, :, :] + pe_ref[...][:, None, :]


@jax.jit
def kernel(tokens, glove_table, W, b):
    S = _SEQ

    toks2 = tokens.reshape(1, S)

    vectors = pl.pallas_call(
        _gather_body,
        grid=(_VOCAB // _VT,),
        in_specs=[
            pl.BlockSpec(memory_space=pltpu.SMEM),
            pl.BlockSpec((_VT, _GLOVE_DIM), lambda k: (k, 0)),
        ],
        out_specs=pl.BlockSpec((S, _GLOVE_DIM), lambda k: (0, 0)),
        out_shape=jax.ShapeDtypeStruct((S, _GLOVE_DIM), jnp.float32),
    )(toks2, glove_table)

    pe = jnp.asarray(_PE)
    b2 = b.reshape(1, _D_MODEL)

    out = pl.pallas_call(
        _fused_body,
        grid=(S // _TI,),
        in_specs=[
            pl.BlockSpec((S, _GLOVE_DIM), lambda i: (0, 0)),
            pl.BlockSpec((_GLOVE_DIM, _D_MODEL), lambda i: (0, 0)),
            pl.BlockSpec((1, _D_MODEL), lambda i: (0, 0)),
            pl.BlockSpec((_TI, _D_MODEL), lambda i: (i, 0)),
        ],
        out_specs=pl.BlockSpec((_TI, S, _D_MODEL), lambda i: (i, 0, 0)),
        out_shape=jax.ShapeDtypeStruct((S, S, _D_MODEL), jnp.float32),
        scratch_shapes=[pltpu.VMEM((S, _D_MODEL), jnp.float32)],
    )(vectors, W, b2, pe)

    return out


# speedup vs baseline: 1.3434x; 1.3434x over previous
import math

import jax
import jax.numpy as jnp
import numpy as np
from jax.experimental import pallas as pl
from jax.experimental.pallas import tpu as pltpu

_D_MODEL = 768
_GLOVE_DIM = 300
_SEQ = 200
_TI = 8
_NQ = 8


def _pe_const():
    position = np.arange(0, _SEQ, dtype=np.float32)[:, None]
    div_term = np.exp(
        np.arange(0, _D_MODEL, 2, dtype=np.float32)
        * (-math.log(10000.0) / _D_MODEL)
    )
    pe = np.zeros((_SEQ, _D_MODEL), dtype=np.float32)
    pe[:, 0::2] = np.sin(position * div_term)
    pe[:, 1::2] = np.cos(position * div_term)
    return pe


_PE = _pe_const()


def _fused_body(toks_ref, *refs):
    glove_refs = refs[:_NQ]
    w_ref, b_ref, pe_ref, out_ref, vec_ref, y_ref, sem = refs[_NQ:]
    i = pl.program_id(0)

    @pl.when(i == 0)
    def _():
        copies = [
            pltpu.make_async_copy(
                glove_refs[j % _NQ].at[pl.ds(toks_ref[0, j], 1)],
                vec_ref.at[pl.ds(j, 1)],
                sem.at[j % _NQ],
            )
            for j in range(_SEQ)
        ]
        for c in copies:
            c.start()
        for c in copies:
            c.wait()
        y_ref[...] = (
            jnp.dot(vec_ref[...], w_ref[...], preferred_element_type=jnp.float32)
            + b_ref[...]
        )

    out_ref[...] = y_ref[...][None, :, :] + pe_ref[...][:, None, :]


@jax.jit
def kernel(tokens, glove_table, W, b):
    S = _SEQ

    pe = jnp.asarray(_PE)
    b2 = b.reshape(1, _D_MODEL)
    toks2 = tokens.reshape(1, S)

    out = pl.pallas_call(
        _fused_body,
        grid=(S // _TI,),
        in_specs=[pl.BlockSpec(memory_space=pltpu.SMEM)]
        + [pl.BlockSpec(memory_space=pltpu.HBM)] * _NQ
        + [
            pl.BlockSpec((_GLOVE_DIM, _D_MODEL), lambda i: (0, 0)),
            pl.BlockSpec((1, _D_MODEL), lambda i: (0, 0)),
            pl.BlockSpec((_TI, _D_MODEL), lambda i: (i, 0)),
        ],
        out_specs=pl.BlockSpec((_TI, S, _D_MODEL), lambda i: (i, 0, 0)),
        out_shape=jax.ShapeDtypeStruct((S, S, _D_MODEL), jnp.float32),
        scratch_shapes=[
            pltpu.VMEM((S, _GLOVE_DIM), jnp.float32),
            pltpu.VMEM((S, _D_MODEL), jnp.float32),
            pltpu.SemaphoreType.DMA((_NQ,)),
        ],
    )(toks2, *([glove_table] * _NQ), W, b2, pe)

    return out
